# baseline (speedup 1.0000x reference)
import jax
import jax.numpy as jnp
from jax.experimental import pallas as pl
from jax.experimental.pallas import tpu as pltpu

_ROW_BLOCK = 1600


def _fused_heads_kernel(x_ref, w_ref, b_ref, o1_ref, o2_ref):
    acc = b_ref[...] + jnp.dot(x_ref[...], w_ref[...],
                               preferred_element_type=jnp.float32)
    o1_ref[...] = acc[:, : o1_ref.shape[1]]
    o2_ref[...] = acc[:, 128 : 128 + o2_ref.shape[1]]


def kernel(x, W_cls, b_cls, W_box, b_box):
    if x.ndim > 2:
        x = x.reshape(x.shape[0], -1)
    R, D = x.shape
    n1 = W_cls.shape[1]
    n2 = W_box.shape[1]

    W = jnp.concatenate(
        [jnp.pad(W_cls, ((0, 0), (0, 128 - n1))),
         jnp.pad(W_box, ((0, 0), (0, 128 - n2)))], axis=1)
    b = jnp.concatenate(
        [jnp.pad(b_cls, (0, 128 - n1)), jnp.pad(b_box, (0, 128 - n2))]
    ).reshape(1, 256)

    o1, o2 = pl.pallas_call(
        _fused_heads_kernel,
        grid=(pl.cdiv(R, _ROW_BLOCK),),
        in_specs=[
            pl.BlockSpec((_ROW_BLOCK, D), lambda i: (i, 0)),
            pl.BlockSpec((D, 256), lambda i: (0, 0)),
            pl.BlockSpec((1, 256), lambda i: (0, 0)),
        ],
        out_specs=[
            pl.BlockSpec((_ROW_BLOCK, n1), lambda i: (i, 0)),
            pl.BlockSpec((_ROW_BLOCK, n2), lambda i: (i, 0)),
        ],
        out_shape=[
            jax.ShapeDtypeStruct((R, n1), jnp.float32),
            jax.ShapeDtypeStruct((R, n2), jnp.float32),
        ],
        compiler_params=pltpu.CompilerParams(
            dimension_semantics=("parallel",),
            vmem_limit_bytes=64 * 1024 * 1024,
        ),
    )(x, W, b)

    return o1, o2


# speedup vs baseline: 1.0132x; 1.0132x over previous
import jax
import jax.numpy as jnp
from jax.experimental import pallas as pl
from jax.experimental.pallas import tpu as pltpu

_ROW_BLOCK = 1000


def _fused_heads_kernel(x_ref, w_ref, b_ref, o1_ref, o2_ref):
    acc = b_ref[...] + jnp.dot(x_ref[...], w_ref[...],
                               preferred_element_type=jnp.float32)
    o1_ref[...] = acc[:, : o1_ref.shape[1]]
    o2_ref[...] = acc[:, 128 : 128 + o2_ref.shape[1]]


def kernel(x, W_cls, b_cls, W_box, b_box):
    if x.ndim > 2:
        x = x.reshape(x.shape[0], -1)
    R, D = x.shape
    n1 = W_cls.shape[1]
    n2 = W_box.shape[1]

    W = jnp.concatenate(
        [jnp.pad(W_cls, ((0, 0), (0, 128 - n1))),
         jnp.pad(W_box, ((0, 0), (0, 128 - n2)))], axis=1)
    b = jnp.concatenate(
        [jnp.pad(b_cls, (0, 128 - n1)), jnp.pad(b_box, (0, 128 - n2))]
    ).reshape(1, 256)

    o1, o2 = pl.pallas_call(
        _fused_heads_kernel,
        grid=(pl.cdiv(R, _ROW_BLOCK),),
        in_specs=[
            pl.BlockSpec((_ROW_BLOCK, D), lambda i: (i, 0)),
            pl.BlockSpec((D, 256), lambda i: (0, 0)),
            pl.BlockSpec((1, 256), lambda i: (0, 0)),
        ],
        out_specs=[
            pl.BlockSpec((_ROW_BLOCK, n1), lambda i: (i, 0)),
            pl.BlockSpec((_ROW_BLOCK, n2), lambda i: (i, 0)),
        ],
        out_shape=[
            jax.ShapeDtypeStruct((R, n1), jnp.float32),
            jax.ShapeDtypeStruct((R, n2), jnp.float32),
        ],
        compiler_params=pltpu.CompilerParams(
            dimension_semantics=("parallel",),
        ),
    )(x, W, b)

    return o1, o2


# speedup vs baseline: 1.0181x; 1.0049x over previous
import jax
import jax.numpy as jnp
from jax.experimental import pallas as pl
from jax.experimental.pallas import tpu as pltpu

_ROW_BLOCK = 1000


def _fused_heads_kernel(x_ref, wc_ref, bc_ref, wb_ref, bb_ref,
                        o1_ref, o2_ref, wp, bp):
    i = pl.program_id(0)
    n1 = wc_ref.shape[1]
    n2 = wb_ref.shape[1]
    D = wc_ref.shape[0]

    @pl.when(i == 0)
    def _pack():
        wp[:, :n1] = wc_ref[...]
        wp[:, n1:128] = jnp.zeros((D, 128 - n1), jnp.float32)
        wp[:, 128:128 + n2] = wb_ref[...]
        wp[:, 128 + n2:] = jnp.zeros((D, 128 - n2), jnp.float32)
        bp[:, :n1] = bc_ref[...]
        bp[:, n1:128] = jnp.zeros((1, 128 - n1), jnp.float32)
        bp[:, 128:128 + n2] = bb_ref[...]
        bp[:, 128 + n2:] = jnp.zeros((1, 128 - n2), jnp.float32)

    acc = bp[...] + jnp.dot(x_ref[...], wp[...],
                            preferred_element_type=jnp.float32)
    o1_ref[...] = acc[:, :n1]
    o2_ref[...] = acc[:, 128:128 + n2]


def kernel(x, W_cls, b_cls, W_box, b_box):
    if x.ndim > 2:
        x = x.reshape(x.shape[0], -1)
    R, D = x.shape
    n1 = W_cls.shape[1]
    n2 = W_box.shape[1]

    o1, o2 = pl.pallas_call(
        _fused_heads_kernel,
        grid=(pl.cdiv(R, _ROW_BLOCK),),
        in_specs=[
            pl.BlockSpec((_ROW_BLOCK, D), lambda i: (i, 0)),
            pl.BlockSpec((D, n1), lambda i: (0, 0)),
            pl.BlockSpec((1, n1), lambda i: (0, 0)),
            pl.BlockSpec((D, n2), lambda i: (0, 0)),
            pl.BlockSpec((1, n2), lambda i: (0, 0)),
        ],
        out_specs=[
            pl.BlockSpec((_ROW_BLOCK, n1), lambda i: (i, 0)),
            pl.BlockSpec((_ROW_BLOCK, n2), lambda i: (i, 0)),
        ],
        out_shape=[
            jax.ShapeDtypeStruct((R, n1), jnp.float32),
            jax.ShapeDtypeStruct((R, n2), jnp.float32),
        ],
        scratch_shapes=[
            pltpu.VMEM((D, 256), jnp.float32),
            pltpu.VMEM((1, 256), jnp.float32),
        ],
    )(x, W_cls, b_cls.reshape(1, n1), W_box, b_box.reshape(1, n2))

    return o1, o2
